# initial kernel scaffold (unmeasured)
import jax
import jax.numpy as jnp
from jax import lax
from jax.experimental import pallas as pl
from jax.experimental.pallas import tpu as pltpu

N_DEV = 4
SQ = 256
SKV = 4096
H_PER = 8
DH = 128
D_MODEL = 1024
SCALE = 0.08838834764831843


def kernel(x, Wq, K_ext, V_ext, Wo):
    def body(x_ref, wq_ref, k_ref, v_ref, wo_ref, out_ref,
             comm_ref, send_sems, recv_sems):
        my_pos = lax.axis_index("i")
        left = (my_pos - 1) % N_DEV
        right = (my_pos + 1) % N_DEV

        barrier_sem = pltpu.get_barrier_semaphore()
        for nbr in [left, right]:
            pl.semaphore_signal(
                barrier_sem, inc=1,
                device_id=(nbr,), device_id_type=pl.DeviceIdType.MESH,
            )
        pl.semaphore_wait(barrier_sem, 2)

        qb = lax.broadcasted_iota(jnp.int32, (SQ, SKV), 0) // 64
        kb = lax.broadcasted_iota(jnp.int32, (SQ, SKV), 1) // 64
        mask = (qb == kb) | (kb == 0) | ((qb + kb) % 3 == 0)

        xm = x_ref[0]
        col0 = my_pos * (H_PER * DH)

        acc = jnp.zeros((SQ, D_MODEL), dtype=jnp.float32)
        for h in range(H_PER):
            wq_h = wq_ref[:, pl.ds(col0 + h * DH, DH)]
            qh = jnp.dot(xm, wq_h, preferred_element_type=jnp.float32)
            kh = k_ref[0, :, h, :]
            vh = v_ref[0, :, h, :]
            s = lax.dot_general(
                qh, kh, (((1,), (1,)), ((), ())),
                preferred_element_type=jnp.float32,
            ) * SCALE
            s = jnp.where(mask, s, -1e9)
            m = jnp.max(s, axis=1, keepdims=True)
            p = jnp.exp(s - m)
            w = p / jnp.sum(p, axis=1, keepdims=True)
            ctx = jnp.dot(w, vh, preferred_element_type=jnp.float32)
            wo_h = wo_ref[pl.ds(col0 + h * DH, DH), :]
            acc = acc + jnp.dot(ctx, wo_h, preferred_element_type=jnp.float32)

        comm_ref[0] = acc
        for h in range(N_DEV - 1):
            rdma = pltpu.make_async_remote_copy(
                src_ref=comm_ref.at[h],
                dst_ref=comm_ref.at[h + 1],
                send_sem=send_sems.at[h],
                recv_sem=recv_sems.at[h + 1],
                device_id=(right,),
                device_id_type=pl.DeviceIdType.MESH,
            )
            rdma.start()
            rdma.wait()
            acc = acc + comm_ref[h + 1]

        out_ref[0] = acc

    out_shape = jax.ShapeDtypeStruct((1, SQ, D_MODEL), jnp.float32)
    return pl.pallas_call(
        body,
        out_shape=out_shape,
        in_specs=[pl.BlockSpec(memory_space=pltpu.VMEM)] * 5,
        out_specs=pl.BlockSpec(memory_space=pltpu.VMEM),
        scratch_shapes=[
            pltpu.VMEM((N_DEV, SQ, D_MODEL), jnp.float32),
            pltpu.SemaphoreType.DMA((N_DEV,)),
            pltpu.SemaphoreType.DMA((N_DEV,)),
        ],
        compiler_params=pltpu.CompilerParams(collective_id=0),
    )(x, Wq, K_ext, V_ext, Wo)


# baseline (device time: 101605 ns/iter reference)
import jax
import jax.numpy as jnp
from jax import lax
from jax.experimental import pallas as pl
from jax.experimental.pallas import tpu as pltpu

N_DEV = 4
SQ = 256
SKV = 4096
H_PER = 8
DH = 128
D_MODEL = 1024
SCALE = 0.08838834764831843


def kernel(x, Wq, K_ext, V_ext, Wo):
    def body(x_ref, wq_hbm, k_hbm, v_hbm, wo_hbm, out_ref,
             wq_buf, k_buf, v_buf, wo_buf, comm_ref,
             load_sems, send_sems, recv_sems):
        my_pos = lax.axis_index("i")
        left = (my_pos - 1) % N_DEV
        right = (my_pos + 1) % N_DEV

        barrier_sem = pltpu.get_barrier_semaphore()
        for nbr in [left, right]:
            pl.semaphore_signal(
                barrier_sem, inc=1,
                device_id=(nbr,), device_id_type=pl.DeviceIdType.MESH,
            )
        pl.semaphore_wait(barrier_sem, 2)

        qb = lax.broadcasted_iota(jnp.int32, (SQ, SKV), 0) // 64
        kb = lax.broadcasted_iota(jnp.int32, (SQ, SKV), 1) // 64
        mask = (qb == kb) | (kb == 0) | ((qb + kb) % 3 == 0)

        xm = x_ref[0]
        col0 = my_pos * (H_PER * DH)

        def load_head(h):
            cps = [
                pltpu.make_async_copy(
                    wq_hbm.at[:, pl.ds(col0 + h * DH, DH)], wq_buf,
                    load_sems.at[0]),
                pltpu.make_async_copy(
                    k_hbm.at[0, :, h, :], k_buf, load_sems.at[1]),
                pltpu.make_async_copy(
                    v_hbm.at[0, :, h, :], v_buf, load_sems.at[2]),
                pltpu.make_async_copy(
                    wo_hbm.at[pl.ds(col0 + h * DH, DH), :], wo_buf,
                    load_sems.at[3]),
            ]
            for cp in cps:
                cp.start()
            for cp in cps:
                cp.wait()

        acc = jnp.zeros((SQ, D_MODEL), dtype=jnp.float32)
        for h in range(H_PER):
            load_head(h)
            qh = jnp.dot(xm, wq_buf[...], preferred_element_type=jnp.float32)
            s = lax.dot_general(
                qh, k_buf[...], (((1,), (1,)), ((), ())),
                preferred_element_type=jnp.float32,
            ) * SCALE
            s = jnp.where(mask, s, -1e9)
            m = jnp.max(s, axis=1, keepdims=True)
            p = jnp.exp(s - m)
            w = p / jnp.sum(p, axis=1, keepdims=True)
            ctx = jnp.dot(w, v_buf[...], preferred_element_type=jnp.float32)
            acc = acc + jnp.dot(ctx, wo_buf[...],
                                preferred_element_type=jnp.float32)

        comm_ref[0] = acc
        for h in range(N_DEV - 1):
            rdma = pltpu.make_async_remote_copy(
                src_ref=comm_ref.at[h],
                dst_ref=comm_ref.at[h + 1],
                send_sem=send_sems.at[h],
                recv_sem=recv_sems.at[h + 1],
                device_id=(right,),
                device_id_type=pl.DeviceIdType.MESH,
            )
            rdma.start()
            rdma.wait()
            acc = acc + comm_ref[h + 1]

        out_ref[0] = acc

    out_shape = jax.ShapeDtypeStruct((1, SQ, D_MODEL), jnp.float32)
    return pl.pallas_call(
        body,
        out_shape=out_shape,
        in_specs=[
            pl.BlockSpec(memory_space=pltpu.VMEM),
            pl.BlockSpec(memory_space=pl.ANY),
            pl.BlockSpec(memory_space=pl.ANY),
            pl.BlockSpec(memory_space=pl.ANY),
            pl.BlockSpec(memory_space=pl.ANY),
        ],
        out_specs=pl.BlockSpec(memory_space=pltpu.VMEM),
        scratch_shapes=[
            pltpu.VMEM((D_MODEL, DH), jnp.float32),
            pltpu.VMEM((SKV, DH), jnp.float32),
            pltpu.VMEM((SKV, DH), jnp.float32),
            pltpu.VMEM((DH, D_MODEL), jnp.float32),
            pltpu.VMEM((N_DEV, SQ, D_MODEL), jnp.float32),
            pltpu.SemaphoreType.DMA((4,)),
            pltpu.SemaphoreType.DMA((N_DEV,)),
            pltpu.SemaphoreType.DMA((N_DEV,)),
        ],
        compiler_params=pltpu.CompilerParams(
            collective_id=0,
            vmem_limit_bytes=56 * 1024 * 1024,
        ),
    )(x, Wq, K_ext, V_ext, Wo)


# device time: 64701 ns/iter; 1.5704x vs baseline; 1.5704x over previous
import jax
import jax.numpy as jnp
from jax import lax
from jax.experimental import pallas as pl
from jax.experimental.pallas import tpu as pltpu

N_DEV = 4
SQ = 256
SKV = 4096
H_PER = 8
DH = 128
D_MODEL = 1024
HALF = D_MODEL // 2
SCALE = 0.08838834764831843


def kernel(x, Wq, K_ext, V_ext, Wo):
    def body(x_ref, wq_hbm, k_hbm, v_hbm, wo_hbm, out_ref,
             wq_buf, k_buf, v_buf, wo_buf, comm_l, comm_r,
             load_sems, send_l, recv_l, send_r, recv_r):
        my_pos = lax.axis_index("i")
        left = (my_pos - 1) % N_DEV
        right = (my_pos + 1) % N_DEV

        col0 = my_pos * (H_PER * DH)

        def head_copies(h, slot):
            return [
                pltpu.make_async_copy(
                    wq_hbm.at[:, pl.ds(col0 + h * DH, DH)], wq_buf.at[slot],
                    load_sems.at[slot, 0]),
                pltpu.make_async_copy(
                    k_hbm.at[0, :, h, :], k_buf.at[slot],
                    load_sems.at[slot, 1]),
                pltpu.make_async_copy(
                    v_hbm.at[0, :, h, :], v_buf.at[slot],
                    load_sems.at[slot, 2]),
                pltpu.make_async_copy(
                    wo_hbm.at[pl.ds(col0 + h * DH, DH), :], wo_buf.at[slot],
                    load_sems.at[slot, 3]),
            ]

        for cp in head_copies(0, 0):
            cp.start()

        barrier_sem = pltpu.get_barrier_semaphore()
        for nbr in [left, right]:
            pl.semaphore_signal(
                barrier_sem, inc=1,
                device_id=(nbr,), device_id_type=pl.DeviceIdType.MESH,
            )
        pl.semaphore_wait(barrier_sem, 2)

        qb = lax.broadcasted_iota(jnp.int32, (SQ, SKV), 0) // 64
        kb = lax.broadcasted_iota(jnp.int32, (SQ, SKV), 1) // 64
        mask = (qb == kb) | (kb == 0) | ((qb + kb) % 3 == 0)

        xm = x_ref[0]

        acc = jnp.zeros((SQ, D_MODEL), dtype=jnp.float32)
        for h in range(H_PER):
            slot = h % 2
            if h + 1 < H_PER:
                for cp in head_copies(h + 1, 1 - slot):
                    cp.start()
            for cp in head_copies(h, slot):
                cp.wait()
            qh = jnp.dot(xm, wq_buf[slot], preferred_element_type=jnp.float32)
            s = lax.dot_general(
                qh, k_buf[slot], (((1,), (1,)), ((), ())),
                preferred_element_type=jnp.float32,
            ) * SCALE
            s = jnp.where(mask, s, -1e9)
            m = jnp.max(s, axis=1, keepdims=True)
            p = jnp.exp(s - m)
            w = p / jnp.sum(p, axis=1, keepdims=True)
            ctx = jnp.dot(w, v_buf[slot], preferred_element_type=jnp.float32)
            acc = acc + jnp.dot(ctx, wo_buf[slot],
                                preferred_element_type=jnp.float32)

        acc_l = acc[:, :HALF]
        acc_r = acc[:, HALF:]
        comm_l[0] = acc_l
        comm_r[0] = acc_r
        for h in range(N_DEV - 1):
            rdma_l = pltpu.make_async_remote_copy(
                src_ref=comm_l.at[h], dst_ref=comm_l.at[h + 1],
                send_sem=send_l.at[h], recv_sem=recv_l.at[h + 1],
                device_id=(right,), device_id_type=pl.DeviceIdType.MESH,
            )
            rdma_r = pltpu.make_async_remote_copy(
                src_ref=comm_r.at[h], dst_ref=comm_r.at[h + 1],
                send_sem=send_r.at[h], recv_sem=recv_r.at[h + 1],
                device_id=(left,), device_id_type=pl.DeviceIdType.MESH,
            )
            rdma_l.start()
            rdma_r.start()
            if h > 0:
                acc_l = acc_l + comm_l[h]
                acc_r = acc_r + comm_r[h]
            rdma_l.wait()
            rdma_r.wait()
        acc_l = acc_l + comm_l[N_DEV - 1]
        acc_r = acc_r + comm_r[N_DEV - 1]

        out_ref[0] = jnp.concatenate([acc_l, acc_r], axis=1)

    out_shape = jax.ShapeDtypeStruct((1, SQ, D_MODEL), jnp.float32)
    return pl.pallas_call(
        body,
        out_shape=out_shape,
        in_specs=[
            pl.BlockSpec(memory_space=pltpu.VMEM),
            pl.BlockSpec(memory_space=pl.ANY),
            pl.BlockSpec(memory_space=pl.ANY),
            pl.BlockSpec(memory_space=pl.ANY),
            pl.BlockSpec(memory_space=pl.ANY),
        ],
        out_specs=pl.BlockSpec(memory_space=pltpu.VMEM),
        scratch_shapes=[
            pltpu.VMEM((2, D_MODEL, DH), jnp.float32),
            pltpu.VMEM((2, SKV, DH), jnp.float32),
            pltpu.VMEM((2, SKV, DH), jnp.float32),
            pltpu.VMEM((2, DH, D_MODEL), jnp.float32),
            pltpu.VMEM((N_DEV, SQ, HALF), jnp.float32),
            pltpu.VMEM((N_DEV, SQ, HALF), jnp.float32),
            pltpu.SemaphoreType.DMA((2, 4)),
            pltpu.SemaphoreType.DMA((N_DEV,)),
            pltpu.SemaphoreType.DMA((N_DEV,)),
            pltpu.SemaphoreType.DMA((N_DEV,)),
            pltpu.SemaphoreType.DMA((N_DEV,)),
        ],
        compiler_params=pltpu.CompilerParams(
            collective_id=0,
            vmem_limit_bytes=56 * 1024 * 1024,
        ),
    )(x, Wq, K_ext, V_ext, Wo)


# device time: 62548 ns/iter; 1.6244x vs baseline; 1.0344x over previous
import jax
import jax.numpy as jnp
from jax import lax
from jax.experimental import pallas as pl
from jax.experimental.pallas import tpu as pltpu

N_DEV = 4
SQ = 256
SKV = 4096
H_PER = 8
DH = 128
D_MODEL = 1024
HALF = D_MODEL // 2
SCALE = 0.08838834764831843

BLK = 64
N_KB = SKV // BLK

def _kept(qb):
    return sorted({0, qb} | {kb for kb in range(N_KB) if (qb + kb) % 3 == 0})

_GROUPS = [
    ([(0, 64), (192, 64)], _kept(0)),
    ([(64, 64)], _kept(1)),
    ([(128, 64)], _kept(2)),
]
_G_NKV = [len(b) * BLK for _, b in _GROUPS]
_G_OFF = [sum(_G_NKV[:g]) for g in range(len(_GROUPS))]
_G_TOT = sum(_G_NKV)
assert _kept(0) == _kept(3)


def kernel(x, Wq, K_ext, V_ext, Wo):
    def body(x_ref, wq_hbm, k_hbm, v_hbm, wo_hbm, out_ref,
             wq_buf, k_buf, v_buf, wo_buf, comm_l, comm_r,
             load_sems, send_l, recv_l, send_r, recv_r):
        my_pos = lax.axis_index("i")
        left = (my_pos - 1) % N_DEV
        right = (my_pos + 1) % N_DEV

        col0 = my_pos * (H_PER * DH)

        def head_copies(h, slot):
            cps = [
                pltpu.make_async_copy(
                    wq_hbm.at[:, pl.ds(col0 + h * DH, DH)], wq_buf.at[slot],
                    load_sems.at[slot, 0]),
                pltpu.make_async_copy(
                    wo_hbm.at[pl.ds(col0 + h * DH, DH), :], wo_buf.at[slot],
                    load_sems.at[slot, 3]),
            ]
            for g, (_, blocks) in enumerate(_GROUPS):
                for j, kbk in enumerate(blocks):
                    dst = _G_OFF[g] + j * BLK
                    cps.append(pltpu.make_async_copy(
                        k_hbm.at[0, pl.ds(kbk * BLK, BLK), h, :],
                        k_buf.at[slot, pl.ds(dst, BLK), :],
                        load_sems.at[slot, 1]))
                    cps.append(pltpu.make_async_copy(
                        v_hbm.at[0, pl.ds(kbk * BLK, BLK), h, :],
                        v_buf.at[slot, pl.ds(dst, BLK), :],
                        load_sems.at[slot, 2]))
            return cps

        for cp in head_copies(0, 0):
            cp.start()

        barrier_sem = pltpu.get_barrier_semaphore()
        for nbr in [left, right]:
            pl.semaphore_signal(
                barrier_sem, inc=1,
                device_id=(nbr,), device_id_type=pl.DeviceIdType.MESH,
            )
        pl.semaphore_wait(barrier_sem, 2)

        xm = x_ref[0]

        acc = jnp.zeros((SQ, D_MODEL), dtype=jnp.float32)
        for h in range(H_PER):
            slot = h % 2
            if h + 1 < H_PER:
                for cp in head_copies(h + 1, 1 - slot):
                    cp.start()
            for cp in head_copies(h, slot):
                cp.wait()
            qh = jnp.dot(xm, wq_buf[slot], preferred_element_type=jnp.float32)
            ctx_parts = {}
            for g, (rows, _) in enumerate(_GROUPS):
                qg = jnp.concatenate(
                    [qh[r0:r0 + n] for r0, n in rows], axis=0)
                kg = k_buf[slot, _G_OFF[g]:_G_OFF[g] + _G_NKV[g], :]
                vg = v_buf[slot, _G_OFF[g]:_G_OFF[g] + _G_NKV[g], :]
                s = lax.dot_general(
                    qg, kg, (((1,), (1,)), ((), ())),
                    preferred_element_type=jnp.float32,
                ) * SCALE
                m = jnp.max(s, axis=1, keepdims=True)
                p = jnp.exp(s - m)
                w = p / jnp.sum(p, axis=1, keepdims=True)
                ctx_parts[g] = jnp.dot(
                    w, vg, preferred_element_type=jnp.float32)
            ctx = jnp.concatenate([
                ctx_parts[0][0:64],
                ctx_parts[1],
                ctx_parts[2],
                ctx_parts[0][64:128],
            ], axis=0)
            acc = acc + jnp.dot(ctx, wo_buf[slot],
                                preferred_element_type=jnp.float32)

        acc_l = acc[:, :HALF]
        acc_r = acc[:, HALF:]
        comm_l[0] = acc_l
        comm_r[0] = acc_r
        for h in range(N_DEV - 1):
            rdma_l = pltpu.make_async_remote_copy(
                src_ref=comm_l.at[h], dst_ref=comm_l.at[h + 1],
                send_sem=send_l.at[h], recv_sem=recv_l.at[h + 1],
                device_id=(right,), device_id_type=pl.DeviceIdType.MESH,
            )
            rdma_r = pltpu.make_async_remote_copy(
                src_ref=comm_r.at[h], dst_ref=comm_r.at[h + 1],
                send_sem=send_r.at[h], recv_sem=recv_r.at[h + 1],
                device_id=(left,), device_id_type=pl.DeviceIdType.MESH,
            )
            rdma_l.start()
            rdma_r.start()
            if h > 0:
                acc_l = acc_l + comm_l[h]
                acc_r = acc_r + comm_r[h]
            rdma_l.wait()
            rdma_r.wait()
        acc_l = acc_l + comm_l[N_DEV - 1]
        acc_r = acc_r + comm_r[N_DEV - 1]

        out_ref[0] = jnp.concatenate([acc_l, acc_r], axis=1)

    out_shape = jax.ShapeDtypeStruct((1, SQ, D_MODEL), jnp.float32)
    return pl.pallas_call(
        body,
        out_shape=out_shape,
        in_specs=[
            pl.BlockSpec(memory_space=pltpu.VMEM),
            pl.BlockSpec(memory_space=pl.ANY),
            pl.BlockSpec(memory_space=pl.ANY),
            pl.BlockSpec(memory_space=pl.ANY),
            pl.BlockSpec(memory_space=pl.ANY),
        ],
        out_specs=pl.BlockSpec(memory_space=pltpu.VMEM),
        scratch_shapes=[
            pltpu.VMEM((2, D_MODEL, DH), jnp.float32),
            pltpu.VMEM((2, _G_TOT, DH), jnp.float32),
            pltpu.VMEM((2, _G_TOT, DH), jnp.float32),
            pltpu.VMEM((2, DH, D_MODEL), jnp.float32),
            pltpu.VMEM((N_DEV, SQ, HALF), jnp.float32),
            pltpu.VMEM((N_DEV, SQ, HALF), jnp.float32),
            pltpu.SemaphoreType.DMA((2, 4)),
            pltpu.SemaphoreType.DMA((N_DEV,)),
            pltpu.SemaphoreType.DMA((N_DEV,)),
            pltpu.SemaphoreType.DMA((N_DEV,)),
            pltpu.SemaphoreType.DMA((N_DEV,)),
        ],
        compiler_params=pltpu.CompilerParams(
            collective_id=0,
            vmem_limit_bytes=56 * 1024 * 1024,
        ),
    )(x, Wq, K_ext, V_ext, Wo)
